# baseline (speedup 1.0000x reference)
import functools

import jax
import jax.numpy as jnp
from jax import lax
from jax.experimental import pallas as pl
from jax.experimental.pallas import tpu as pltpu
from jax.experimental.pallas import tpu_sc as plsc

T = 64
H, W = 361, 720
SLAB = H * W
NWORKERS = 32
CH = 8128
LAST_BASE = SLAB - CH

_MESH = plsc.VectorSubcoreMesh(core_axis_name="c", subcore_axis_name="s")


@functools.partial(
    pl.kernel,
    mesh=_MESH,
    out_type=[
        jax.ShapeDtypeStruct((SLAB,), jnp.float32),
        jax.ShapeDtypeStruct((SLAB,), jnp.float32),
    ],
    scratch_types=[
        pltpu.VMEM((16,), jnp.int32),
        pltpu.VMEM((CH,), jnp.float32),
        pltpu.VMEM((CH,), jnp.float32),
        pltpu.SemaphoreType.DMA,
        pltpu.SemaphoreType.DMA,
    ],
)
def _dynamic_slice_sc(start_hbm, temp_hbm, wind_hbm, out_t_hbm, out_w_hbm,
                      start_v, buf_t, buf_w, sem_t, sem_w):
    wid = lax.axis_index("s") * 2 + lax.axis_index("c")
    base = jnp.minimum(wid * CH, LAST_BASE)
    pltpu.sync_copy(start_hbm, start_v)
    base = pl.multiple_of(base, 8)
    start_vec = start_v[...]
    start = pl.multiple_of(start_vec[0] + base, 8)
    ct = pltpu.async_copy(temp_hbm.at[pl.ds(start, CH)], buf_t, sem_t)
    cw = pltpu.async_copy(wind_hbm.at[pl.ds(start, CH)], buf_w, sem_w)
    ct.wait()
    st = pltpu.async_copy(buf_t, out_t_hbm.at[pl.ds(base, CH)], sem_t)
    cw.wait()
    sw = pltpu.async_copy(buf_w, out_w_hbm.at[pl.ds(base, CH)], sem_w)
    st.wait()
    sw.wait()


def kernel(time, times, temperature, wind_speed):
    t = time[0]
    time_indices = jnp.arange(times.size, dtype=times.dtype)
    approx_index = jnp.interp(t, times, time_indices)
    index = jnp.round(approx_index).astype(jnp.int32)
    start0 = jnp.full((16,), index * SLAB, dtype=jnp.int32)
    out_t, out_w = _dynamic_slice_sc(
        start0,
        temperature.reshape(T * SLAB),
        wind_speed.reshape(T * SLAB),
    )
    return out_t.reshape(H, W), out_w.reshape(H, W)


# speedup vs baseline: 6.4926x; 6.4926x over previous
import functools

import jax
import jax.numpy as jnp
from jax import lax
from jax.experimental import pallas as pl
from jax.experimental.pallas import tpu as pltpu
from jax.experimental.pallas import tpu_sc as plsc

T = 64
H, W = 361, 720

_MESH = plsc.VectorSubcoreMesh(core_axis_name="c", subcore_axis_name="s")


@functools.partial(
    pl.kernel,
    mesh=_MESH,
    out_type=[
        jax.ShapeDtypeStruct((1, H, W), jnp.float32),
        jax.ShapeDtypeStruct((1, H, W), jnp.float32),
    ],
    scratch_types=[
        pltpu.VMEM((16,), jnp.int32),
        pltpu.SemaphoreType.DMA,
    ],
    compiler_params=pltpu.CompilerParams(use_tc_tiling_on_sc=True),
)
def _dynamic_slice_sc(idx_hbm, temp_hbm, wind_hbm, out_t_hbm, out_w_hbm,
                      idx_v, sem):
    cid = lax.axis_index("c")
    sid = lax.axis_index("s")

    @pl.when((sid == 0) & (cid == 0))
    def _copy_temp():
        pltpu.sync_copy(idx_hbm, idx_v)
        idx = idx_v[...][0]
        pltpu.async_copy(temp_hbm.at[pl.ds(idx, 1)], out_t_hbm, sem).wait()

    @pl.when((sid == 0) & (cid == 1))
    def _copy_wind():
        pltpu.sync_copy(idx_hbm, idx_v)
        idx = idx_v[...][0]
        pltpu.async_copy(wind_hbm.at[pl.ds(idx, 1)], out_w_hbm, sem).wait()


def kernel(time, times, temperature, wind_speed):
    t = time[0]
    time_indices = jnp.arange(times.size, dtype=times.dtype)
    approx_index = jnp.interp(t, times, time_indices)
    index = jnp.round(approx_index).astype(jnp.int32)
    idx_arr = jnp.full((16,), index, dtype=jnp.int32)
    out_t, out_w = _dynamic_slice_sc(idx_arr, temperature, wind_speed)
    return out_t.reshape(H, W), out_w.reshape(H, W)


# speedup vs baseline: 18.3435x; 2.8253x over previous
import functools

import jax
import jax.numpy as jnp
from jax import lax
from jax.experimental import pallas as pl
from jax.experimental.pallas import tpu as pltpu
from jax.experimental.pallas import tpu_sc as plsc

T = 64
H, W = 361, 720
NCHUNK = 15
ROWS = W // NCHUNK

_MESH = plsc.VectorSubcoreMesh(core_axis_name="c", subcore_axis_name="s")


@functools.partial(
    pl.kernel,
    mesh=_MESH,
    out_type=[
        jax.ShapeDtypeStruct((1, W, H), jnp.float32),
        jax.ShapeDtypeStruct((1, W, H), jnp.float32),
    ],
    scratch_types=[
        pltpu.VMEM((16,), jnp.int32),
        pltpu.SemaphoreType.DMA,
    ],
)
def _dynamic_slice_sc(idx_hbm, temp_hbm, wind_hbm, out_t_hbm, out_w_hbm,
                      idx_v, sem):
    cid = lax.axis_index("c")
    sid = lax.axis_index("s")
    wid = sid * 2 + cid
    field = wid % 2
    g = wid // 2
    base = g * ROWS

    @pl.when(g < NCHUNK)
    def _copy_chunk():
        pltpu.sync_copy(idx_hbm, idx_v)
        idx = idx_v[...][0]

        @pl.when(field == 0)
        def _temp():
            pltpu.async_copy(
                temp_hbm.at[pl.ds(idx, 1), pl.ds(base, ROWS), :],
                out_t_hbm.at[:, pl.ds(base, ROWS), :],
                sem,
            ).wait()

        @pl.when(field == 1)
        def _wind():
            pltpu.async_copy(
                wind_hbm.at[pl.ds(idx, 1), pl.ds(base, ROWS), :],
                out_w_hbm.at[:, pl.ds(base, ROWS), :],
                sem,
            ).wait()


def kernel(time, times, temperature, wind_speed):
    t = time[0]
    n = times.size
    cnt = jnp.sum((times <= t).astype(jnp.int32))
    j = jnp.clip(cnt - 1, 0, n - 2)
    t0 = lax.dynamic_index_in_dim(times, j, keepdims=False)
    t1 = lax.dynamic_index_in_dim(times, j + 1, keepdims=False)
    approx = j.astype(jnp.float32) + (t - t0) / (t1 - t0)
    approx = jnp.clip(approx, 0.0, jnp.float32(n - 1))
    index = jnp.round(approx).astype(jnp.int32)

    idx_arr = jnp.full((16,), index, dtype=jnp.int32)
    out_t, out_w = _dynamic_slice_sc(
        idx_arr,
        jnp.transpose(temperature, (0, 2, 1)),
        jnp.transpose(wind_speed, (0, 2, 1)),
    )
    return (
        jnp.transpose(out_t.reshape(W, H)),
        jnp.transpose(out_w.reshape(W, H)),
    )


# speedup vs baseline: 60.3409x; 3.2895x over previous
import functools

import jax
import jax.numpy as jnp
from jax import lax
from jax.experimental import pallas as pl
from jax.experimental.pallas import tpu as pltpu
from jax.experimental.pallas import tpu_sc as plsc

T = 64
H, W = 361, 720
NCHUNK = 15
ROWS = W // NCHUNK

_MESH = plsc.VectorSubcoreMesh(core_axis_name="c", subcore_axis_name="s")


@functools.partial(
    pl.kernel,
    mesh=_MESH,
    out_type=[
        jax.ShapeDtypeStruct((1, W, H), jnp.float32),
        jax.ShapeDtypeStruct((1, W, H), jnp.float32),
    ],
    scratch_types=[
        pltpu.VMEM((16,), jnp.int32),
        pltpu.VMEM((1, ROWS, H), jnp.float32),
        pltpu.SemaphoreType.DMA,
    ],
)
def _dynamic_slice_sc(idx_hbm, temp_hbm, wind_hbm, out_t_hbm, out_w_hbm,
                      idx_v, buf, sem):
    cid = lax.axis_index("c")
    sid = lax.axis_index("s")
    wid = sid * 2 + cid
    field = wid % 2
    g = wid // 2
    base = g * ROWS

    @pl.when(g < NCHUNK)
    def _copy_chunk():
        pltpu.sync_copy(idx_hbm, idx_v)
        idx = idx_v[...][0]

        @pl.when(field == 0)
        def _temp():
            pltpu.async_copy(
                temp_hbm.at[pl.ds(idx, 1), pl.ds(base, ROWS), :], buf, sem
            ).wait()
            pltpu.async_copy(
                buf, out_t_hbm.at[:, pl.ds(base, ROWS), :], sem
            ).wait()

        @pl.when(field == 1)
        def _wind():
            pltpu.async_copy(
                wind_hbm.at[pl.ds(idx, 1), pl.ds(base, ROWS), :], buf, sem
            ).wait()
            pltpu.async_copy(
                buf, out_w_hbm.at[:, pl.ds(base, ROWS), :], sem
            ).wait()


def kernel(time, times, temperature, wind_speed):
    t = time[0]
    n = times.size
    cnt = jnp.sum((times <= t).astype(jnp.int32))
    j = jnp.clip(cnt - 1, 0, n - 2)
    t0 = lax.dynamic_index_in_dim(times, j, keepdims=False)
    t1 = lax.dynamic_index_in_dim(times, j + 1, keepdims=False)
    approx = j.astype(jnp.float32) + (t - t0) / (t1 - t0)
    approx = jnp.clip(approx, 0.0, jnp.float32(n - 1))
    index = jnp.round(approx).astype(jnp.int32)

    idx_arr = jnp.full((16,), index, dtype=jnp.int32)
    out_t, out_w = _dynamic_slice_sc(
        idx_arr,
        jnp.transpose(temperature, (0, 2, 1)),
        jnp.transpose(wind_speed, (0, 2, 1)),
    )
    return (
        jnp.transpose(out_t.reshape(W, H)),
        jnp.transpose(out_w.reshape(W, H)),
    )


# speedup vs baseline: 69.6463x; 1.1542x over previous
import functools

import jax
import jax.numpy as jnp
from jax import lax
from jax.experimental import pallas as pl
from jax.experimental.pallas import tpu as pltpu
from jax.experimental.pallas import tpu_sc as plsc

T = 64
H, W = 361, 720
NCHUNK = 15
ROWS = W // NCHUNK
L = 16

_MESH = plsc.VectorSubcoreMesh(core_axis_name="c", subcore_axis_name="s")


@functools.partial(
    pl.kernel,
    mesh=_MESH,
    out_type=[
        jax.ShapeDtypeStruct((1, W, H), jnp.float32),
        jax.ShapeDtypeStruct((1, W, H), jnp.float32),
    ],
    scratch_types=[
        pltpu.VMEM((L,), jnp.float32),
        pltpu.VMEM((T,), jnp.float32),
        pltpu.VMEM((1, ROWS, H), jnp.float32),
        pltpu.SemaphoreType.DMA,
        pltpu.SemaphoreType.DMA,
    ],
    compiler_params=pltpu.CompilerParams(needs_layout_passes=False),
)
def _dynamic_slice_sc(time_hbm, times_hbm, temp_hbm, wind_hbm,
                      out_t_hbm, out_w_hbm,
                      time_v, times_v, buf, sem_a, sem_b):
    cid = lax.axis_index("c")
    sid = lax.axis_index("s")
    wid = sid * 2 + cid
    field = wid % 2
    g = wid // 2
    base = g * ROWS

    @pl.when(g < NCHUNK)
    def _copy_chunk():
        ca = pltpu.make_async_copy(time_hbm, time_v.at[pl.ds(0, 1)], sem_a)
        cb = pltpu.make_async_copy(times_hbm, times_v, sem_b)
        ca.start()
        cb.start()
        ca.wait()
        cb.wait()

        t = time_v[...][0]

        def probe(i):
            return plsc.load_gather(times_v, [jnp.full((L,), i, jnp.int32)])[0]

        cnt = jnp.int32(0)
        for step in (32, 16, 8, 4, 2, 1):
            nxt = cnt + step
            cnt = jnp.where(probe(nxt - 1) <= t, nxt, cnt)
        j = jnp.full((L,), jnp.clip(cnt - 1, 0, T - 2), dtype=jnp.int32)
        t0 = plsc.load_gather(times_v, [j])
        t1 = plsc.load_gather(times_v, [j + 1])
        tv = jnp.full((L,), t, dtype=jnp.float32)
        approx = j.astype(jnp.float32) + (tv - t0) / (t1 - t0)
        approx = jnp.clip(approx, 0.0, jnp.float32(T - 1))
        y = (approx + 0.5).astype(jnp.int32)
        exact_half = (approx + 0.5) == y.astype(jnp.float32)
        idx_v = y - jnp.where(exact_half & ((y % 2) == 1), 1, 0)
        idx = idx_v[0]

        @pl.when(field == 0)
        def _temp():
            pltpu.async_copy(
                temp_hbm.at[pl.ds(idx, 1), pl.ds(base, ROWS), :], buf, sem_a
            ).wait()
            pltpu.async_copy(
                buf, out_t_hbm.at[:, pl.ds(base, ROWS), :], sem_a
            ).wait()

        @pl.when(field == 1)
        def _wind():
            pltpu.async_copy(
                wind_hbm.at[pl.ds(idx, 1), pl.ds(base, ROWS), :], buf, sem_a
            ).wait()
            pltpu.async_copy(
                buf, out_w_hbm.at[:, pl.ds(base, ROWS), :], sem_a
            ).wait()


def kernel(time, times, temperature, wind_speed):
    out_t, out_w = _dynamic_slice_sc(
        time,
        times,
        jnp.transpose(temperature, (0, 2, 1)),
        jnp.transpose(wind_speed, (0, 2, 1)),
    )
    return (
        jnp.transpose(out_t.reshape(W, H)),
        jnp.transpose(out_w.reshape(W, H)),
    )


# speedup vs baseline: 71.6640x; 1.0290x over previous
import functools

import jax
import jax.numpy as jnp
from jax import lax
from jax.experimental import pallas as pl
from jax.experimental.pallas import tpu as pltpu
from jax.experimental.pallas import tpu_sc as plsc

T = 64
H, W = 361, 720
NCHUNK = 15
ROWS = W // NCHUNK
L = 16

_MESH = plsc.VectorSubcoreMesh(
    core_axis_name="c", subcore_axis_name="s", num_cores=1
)


@functools.partial(
    pl.kernel,
    mesh=_MESH,
    out_type=[
        jax.ShapeDtypeStruct((1, W, H), jnp.float32),
        jax.ShapeDtypeStruct((1, W, H), jnp.float32),
    ],
    scratch_types=[
        pltpu.VMEM((L,), jnp.float32),
        pltpu.VMEM((T,), jnp.float32),
        pltpu.VMEM((1, ROWS, H), jnp.float32),
        pltpu.VMEM((1, ROWS, H), jnp.float32),
        pltpu.SemaphoreType.DMA,
        pltpu.SemaphoreType.DMA,
    ],
    compiler_params=pltpu.CompilerParams(needs_layout_passes=False),
)
def _dynamic_slice_sc(time_hbm, times_hbm, temp_hbm, wind_hbm,
                      out_t_hbm, out_w_hbm,
                      time_v, times_v, buf_t, buf_w, sem_a, sem_b):
    sid = lax.axis_index("s")
    base = jnp.minimum(sid * ROWS, W - ROWS)

    if True:
        ca = pltpu.make_async_copy(time_hbm, time_v.at[pl.ds(0, 1)], sem_a)
        cb = pltpu.make_async_copy(times_hbm, times_v, sem_b)
        ca.start()
        cb.start()
        ca.wait()
        cb.wait()

        t = time_v[...][0]

        def probe(i):
            return plsc.load_gather(times_v, [jnp.full((L,), i, jnp.int32)])[0]

        cnt = jnp.int32(0)
        for step in (32, 16, 8, 4, 2, 1):
            nxt = cnt + step
            cnt = jnp.where(probe(nxt - 1) <= t, nxt, cnt)
        j = jnp.full((L,), jnp.clip(cnt - 1, 0, T - 2), dtype=jnp.int32)
        t0 = plsc.load_gather(times_v, [j])
        t1 = plsc.load_gather(times_v, [j + 1])
        tv = jnp.full((L,), t, dtype=jnp.float32)
        approx = j.astype(jnp.float32) + (tv - t0) / (t1 - t0)
        approx = jnp.clip(approx, 0.0, jnp.float32(T - 1))
        y = (approx + 0.5).astype(jnp.int32)
        exact_half = (approx + 0.5) == y.astype(jnp.float32)
        idx_v = y - jnp.where(exact_half & ((y % 2) == 1), 1, 0)
        idx = idx_v[0]

        gt = pltpu.make_async_copy(
            temp_hbm.at[pl.ds(idx, 1), pl.ds(base, ROWS), :], buf_t, sem_a
        )
        gw = pltpu.make_async_copy(
            wind_hbm.at[pl.ds(idx, 1), pl.ds(base, ROWS), :], buf_w, sem_b
        )
        gt.start()
        gw.start()
        gt.wait()
        st = pltpu.make_async_copy(
            buf_t, out_t_hbm.at[:, pl.ds(base, ROWS), :], sem_a
        )
        st.start()
        gw.wait()
        sw = pltpu.make_async_copy(
            buf_w, out_w_hbm.at[:, pl.ds(base, ROWS), :], sem_b
        )
        sw.start()
        st.wait()
        sw.wait()


def kernel(time, times, temperature, wind_speed):
    out_t, out_w = _dynamic_slice_sc(
        time,
        times,
        jnp.transpose(temperature, (0, 2, 1)),
        jnp.transpose(wind_speed, (0, 2, 1)),
    )
    return (
        jnp.transpose(out_t.reshape(W, H)),
        jnp.transpose(out_w.reshape(W, H)),
    )


# speedup vs baseline: 71.9730x; 1.0043x over previous
import functools

import jax
import jax.numpy as jnp
from jax import lax
from jax.experimental import pallas as pl
from jax.experimental.pallas import tpu as pltpu
from jax.experimental.pallas import tpu_sc as plsc

T = 64
H, W = 361, 720
NCHUNK = 15
ROWS = W // NCHUNK
L = 16

_MESH = plsc.VectorSubcoreMesh(
    core_axis_name="c", subcore_axis_name="s", num_cores=1
)


@functools.partial(
    pl.kernel,
    mesh=_MESH,
    out_type=[
        jax.ShapeDtypeStruct((1, W, H), jnp.float32),
        jax.ShapeDtypeStruct((1, W, H), jnp.float32),
    ],
    scratch_types=[
        pltpu.VMEM((L,), jnp.float32),
        pltpu.VMEM((T,), jnp.float32),
        pltpu.VMEM((1, ROWS, H), jnp.float32),
        pltpu.VMEM((1, ROWS, H), jnp.float32),
        pltpu.SemaphoreType.DMA,
        pltpu.SemaphoreType.DMA,
    ],
    compiler_params=pltpu.CompilerParams(
        needs_layout_passes=False, skip_device_barrier=True
    ),
)
def _dynamic_slice_sc(time_hbm, times_hbm, temp_hbm, wind_hbm,
                      out_t_hbm, out_w_hbm,
                      time_v, times_v, buf_t, buf_w, sem_a, sem_b):
    sid = lax.axis_index("s")
    base = jnp.minimum(sid * ROWS, W - ROWS)

    if True:
        ca = pltpu.make_async_copy(time_hbm, time_v.at[pl.ds(0, 1)], sem_a)
        cb = pltpu.make_async_copy(times_hbm, times_v, sem_b)
        ca.start()
        cb.start()
        ca.wait()
        cb.wait()

        t = time_v[...][0]

        def probe(i):
            return plsc.load_gather(times_v, [jnp.full((L,), i, jnp.int32)])[0]

        cnt = jnp.int32(0)
        for step in (32, 16, 8, 4, 2, 1):
            nxt = cnt + step
            cnt = jnp.where(probe(nxt - 1) <= t, nxt, cnt)
        j = jnp.full((L,), jnp.clip(cnt - 1, 0, T - 2), dtype=jnp.int32)
        t0 = plsc.load_gather(times_v, [j])
        t1 = plsc.load_gather(times_v, [j + 1])
        tv = jnp.full((L,), t, dtype=jnp.float32)
        approx = j.astype(jnp.float32) + (tv - t0) / (t1 - t0)
        approx = jnp.clip(approx, 0.0, jnp.float32(T - 1))
        y = (approx + 0.5).astype(jnp.int32)
        exact_half = (approx + 0.5) == y.astype(jnp.float32)
        idx_v = y - jnp.where(exact_half & ((y % 2) == 1), 1, 0)
        idx = idx_v[0]

        gt = pltpu.make_async_copy(
            temp_hbm.at[pl.ds(idx, 1), pl.ds(base, ROWS), :], buf_t, sem_a
        )
        gw = pltpu.make_async_copy(
            wind_hbm.at[pl.ds(idx, 1), pl.ds(base, ROWS), :], buf_w, sem_b
        )
        gt.start()
        gw.start()
        gt.wait()
        st = pltpu.make_async_copy(
            buf_t, out_t_hbm.at[:, pl.ds(base, ROWS), :], sem_a
        )
        st.start()
        gw.wait()
        sw = pltpu.make_async_copy(
            buf_w, out_w_hbm.at[:, pl.ds(base, ROWS), :], sem_b
        )
        sw.start()
        st.wait()
        sw.wait()


def kernel(time, times, temperature, wind_speed):
    out_t, out_w = _dynamic_slice_sc(
        time,
        times,
        jnp.transpose(temperature, (0, 2, 1)),
        jnp.transpose(wind_speed, (0, 2, 1)),
    )
    return (
        jnp.transpose(out_t.reshape(W, H)),
        jnp.transpose(out_w.reshape(W, H)),
    )
